# baseline (speedup 1.0000x reference)
import functools

import jax
import jax.numpy as jnp
from jax import lax
from jax.experimental import pallas as pl
from jax.experimental.pallas import tpu as pltpu
from jax.experimental.pallas import tpu_sc as plsc

D = 64
SEQ = 200
BATCH = 4096
NB = 2

_info = plsc.get_sparse_core_info()
_NC = _info.num_cores
_NS = _info.num_subcores
NW = _NC * _NS
BB = BATCH // NW


def _sc_body(q_hbm, t_hbm, wq_hbm, wt_hbm, wp_hbm, out_hbm,
             qblk, tblk, qi, ti, qr, tr, tob, pos_v, gsem, osem):
    wid = lax.axis_index("s") * _NC + lax.axis_index("c")
    b0 = wid * BB

    pltpu.sync_copy(q_hbm.at[pl.ds(b0, BB)], qblk)
    pltpu.sync_copy(t_hbm.at[pl.ds(b0, BB)], tblk)
    pltpu.sync_copy(wp_hbm, pos_v)

    lanes = lax.iota(jnp.int32, 16)

    def extract_idx(s, b):
        col = jnp.full((16,), s, jnp.int32)
        for bg in range(BB // 16):
            rows = lanes + (bg * 16)
            qv = plsc.load_gather(qblk, [rows, col])
            tv = plsc.load_gather(tblk, [rows, col])
            qi[b][pl.ds(bg * 16, 16)] = qv
            ti[b][pl.ds(bg * 16, 16)] = tv

    def fire_gathers(b):
        pltpu.async_copy(wq_hbm.at[qi[b]], qr[b], gsem[b])
        pltpu.async_copy(wt_hbm.at[ti[b]], tr[b], gsem[b])

    def wait_gathers(b):
        pltpu.make_async_copy(wq_hbm.at[qi[b]], qr[b], gsem[b]).wait()
        pltpu.make_async_copy(wt_hbm.at[ti[b]], tr[b], gsem[b]).wait()

    def fire_out(s, b):
        pltpu.async_copy(tob[b], out_hbm.at[s, :, wid], osem[b])

    def wait_out(b):
        pltpu.make_async_copy(tob[b], out_hbm.at[0, :, 0], osem[b]).wait()

    def compute(s, b):
        def body(dg, carry):
            pv16 = pos_v[s, pl.ds(dg * 16, 16)]
            for dd in range(16):
                d = dg * 16 + dd
                tix = dg * 2 + dd // 8
                r = dd % 8
                col = jnp.full((16,), d, jnp.int32)
                pv = jnp.full((16,), pv16[dd], jnp.float32)
                dst = tob[b].at[tix].at[r]
                for bg in range(BB // 16):
                    rows = lanes + (bg * 16)
                    qv = plsc.load_gather(qr[b], [rows, col])
                    tv = plsc.load_gather(tr[b], [rows, col])
                    dst[pl.ds(bg * 16, 16)] = qv + tv + pv
            return carry

        lax.fori_loop(0, D // 16, body, 0)

    def step(s, b, wait_prev_out):
        nb = (b + 1) % NB
        extract_idx(jnp.minimum(s + 1, SEQ - 1), nb)
        if wait_prev_out:
            wait_out(nb)
        fire_gathers(nb)
        wait_gathers(b)
        compute(s, b)
        fire_out(s, b)

    extract_idx(0, 0)
    fire_gathers(0)

    step(0, 0, wait_prev_out=False)

    def loop_body(kk, carry):
        s = 1 + kk * NB
        for b in range(NB):
            step(s + b, (1 + b) % NB, wait_prev_out=True)
        return carry

    n_main = (SEQ - 1) // NB
    lax.fori_loop(0, n_main, loop_body, 0)
    tail_start = 1 + n_main * NB
    for i in range(SEQ - 1 - n_main * NB):
        step(tail_start + i, (1 + i) % NB, wait_prev_out=True)

    b_last = (SEQ - 1) % NB
    nb_last = (b_last + 1) % NB
    wait_gathers(nb_last)
    for i in range(NB - 1):
        wait_out((b_last - i) % NB)


@jax.jit
def _sc_call(q, t, wq, wt, wp):
    mesh = plsc.VectorSubcoreMesh(core_axis_name="c", subcore_axis_name="s")
    run = pl.kernel(
        _sc_body,
        out_type=jax.ShapeDtypeStruct((SEQ, D // 8, NW, 8, 128), jnp.float32),
        mesh=mesh,
        scratch_types=[
            pltpu.VMEM((BB, SEQ), jnp.int32),
            pltpu.VMEM((BB, SEQ), jnp.int32),
            [pltpu.VMEM((BB,), jnp.int32) for _ in range(NB)],
            [pltpu.VMEM((BB,), jnp.int32) for _ in range(NB)],
            [pltpu.VMEM((BB, D), jnp.float32) for _ in range(NB)],
            [pltpu.VMEM((BB, D), jnp.float32) for _ in range(NB)],
            [pltpu.VMEM((D // 8, 8, 128), jnp.float32) for _ in range(NB)],
            pltpu.VMEM((SEQ, D), jnp.float32),
            [pltpu.SemaphoreType.DMA for _ in range(NB)],
            [pltpu.SemaphoreType.DMA for _ in range(NB)],
        ],
        compiler_params=pltpu.CompilerParams(use_tc_tiling_on_sc=False, needs_layout_passes=False),
    )
    return run(q, t, wq, wt, wp)


def kernel(questions, tags, W_question, W_tag, W_pos):
    p5 = _sc_call(questions.astype(jnp.int32), tags.astype(jnp.int32),
                  W_question, W_tag, W_pos)
    return p5.transpose(2, 4, 0, 1, 3).reshape(BATCH, SEQ, D)


# speedup vs baseline: 2.9026x; 2.9026x over previous
import functools

import jax
import jax.numpy as jnp
from jax import lax
from jax.experimental import layout as jlayout
from jax.experimental import pallas as pl
from jax.experimental.pallas import tpu as pltpu
from jax.experimental.pallas import tpu_sc as plsc

D = 64
SEQ = 200
BATCH = 4096
G1 = 128
G2 = SEQ - G1
NB = 2

_info = plsc.get_sparse_core_info()
_NC = _info.num_cores
_NS = _info.num_subcores
NW = _NC * _NS
RPW = BATCH // NW
UNROLL = 4


def _sc_body(q_hbm, t_hbm, wq_hbm, wt_hbm, wp_hbm, out_hbm,
             qi, ti, ob, tr, pos_v, isem, gsem, osem):
    wid = lax.axis_index("s") * _NC + lax.axis_index("c")
    first = wid * RPW
    last = first + RPW - 1

    pltpu.sync_copy(wp_hbm, pos_v)

    def fire_idx(k, b):
        kc = jnp.minimum(k, last)
        pltpu.async_copy(q_hbm.at[kc], qi[b], isem[b])
        pltpu.async_copy(t_hbm.at[kc], ti[b], isem[b])

    def wait_idx(b):
        pltpu.make_async_copy(q_hbm.at[0], qi[b], isem[b]).wait()
        pltpu.make_async_copy(t_hbm.at[0], ti[b], isem[b]).wait()

    def fire_gathers(b):
        pltpu.async_copy(wq_hbm.at[qi[b].at[pl.ds(0, G1)]],
                         ob[b].at[pl.ds(0, G1)], gsem[b])
        pltpu.async_copy(wq_hbm.at[qi[b].at[pl.ds(G1, G2)]],
                         ob[b].at[pl.ds(G1, G2)], gsem[b])
        pltpu.async_copy(wt_hbm.at[ti[b].at[pl.ds(0, G1)]],
                         tr[b].at[pl.ds(0, G1)], gsem[b])
        pltpu.async_copy(wt_hbm.at[ti[b].at[pl.ds(G1, G2)]],
                         tr[b].at[pl.ds(G1, G2)], gsem[b])

    def wait_gathers(b):
        pltpu.make_async_copy(wq_hbm.at[qi[b].at[pl.ds(0, G1)]],
                              ob[b].at[pl.ds(0, G1)], gsem[b]).wait()
        pltpu.make_async_copy(wq_hbm.at[qi[b].at[pl.ds(G1, G2)]],
                              ob[b].at[pl.ds(G1, G2)], gsem[b]).wait()
        pltpu.make_async_copy(wt_hbm.at[ti[b].at[pl.ds(0, G1)]],
                              tr[b].at[pl.ds(0, G1)], gsem[b]).wait()
        pltpu.make_async_copy(wt_hbm.at[ti[b].at[pl.ds(G1, G2)]],
                              tr[b].at[pl.ds(G1, G2)], gsem[b]).wait()

    def fire_out(k, b):
        pltpu.async_copy(ob[b], out_hbm.at[k], osem[b])

    def wait_out(b):
        pltpu.make_async_copy(ob[b], out_hbm.at[0], osem[b]).wait()

    def compute(b):
        def sweep(ii, carry):
            for r in range(UNROLL):
                i = ii * UNROLL + r
                row_t = tr[b].at[i]
                row_o = ob[b].at[i]
                for g in range(D // 16):
                    sl = pl.ds(g * 16, 16)
                    x = row_t[sl] + pos_v[pl.ds(i * D + g * 16, 16)]
                    plsc.addupdate(row_o.at[sl], x)
            return carry

        lax.fori_loop(0, SEQ // UNROLL, sweep, 0)

    def step(k, b, wait_prev_out):
        nb = (b + 1) % NB
        wait_idx(nb)
        if wait_prev_out:
            wait_out(nb)
        fire_gathers(nb)
        wait_gathers(b)
        fire_idx(k + NB, b)
        compute(b)
        fire_out(k, b)

    pltpu.sync_copy(q_hbm.at[first], qi[0])
    pltpu.sync_copy(t_hbm.at[first], ti[0])
    fire_gathers(0)
    fire_idx(first + 1, 1)

    step(first, 0, wait_prev_out=False)

    def loop_body(kk, carry):
        k = first + 1 + kk * NB
        for b in range(NB):
            step(k + b, (1 + b) % NB, wait_prev_out=True)
        return carry

    n_main = (RPW - 1) // NB
    lax.fori_loop(0, n_main, loop_body, 0)
    tail_start = first + 1 + n_main * NB
    for i in range(RPW - 1 - n_main * NB):
        step(tail_start + i, (1 + i) % NB, wait_prev_out=True)

    b_last = (RPW - 1) % NB
    nb_last = (b_last + 1) % NB
    wait_idx(b_last)
    wait_gathers(nb_last)
    for i in range(NB - 1):
        wait_out((b_last - i) % NB)


def _sc_call(q, t, wq, wt, wp):
    mesh = plsc.VectorSubcoreMesh(core_axis_name="c", subcore_axis_name="s")
    run = pl.kernel(
        _sc_body,
        out_type=jax.ShapeDtypeStruct((BATCH, SEQ, D), jnp.float32),
        mesh=mesh,
        scratch_types=[
            [pltpu.VMEM((SEQ,), jnp.int32) for _ in range(NB)],
            [pltpu.VMEM((SEQ,), jnp.int32) for _ in range(NB)],
            [pltpu.VMEM((SEQ, D), jnp.float32) for _ in range(NB)],
            [pltpu.VMEM((SEQ, D), jnp.float32) for _ in range(NB)],
            pltpu.VMEM((SEQ * D,), jnp.float32),
            [pltpu.SemaphoreType.DMA for _ in range(NB)],
            [pltpu.SemaphoreType.DMA for _ in range(NB)],
            [pltpu.SemaphoreType.DMA for _ in range(NB)],
        ],
        compiler_params=pltpu.CompilerParams(use_tc_tiling_on_sc=False),
    )
    return run(q, t, wq, wt, wp)


def _pick_device():
    try:
        mesh = jax.sharding.get_mesh()
        if mesh is not None and getattr(mesh, "devices", None) is not None:
            return mesh.devices.flat[0]
    except Exception:
        pass
    try:
        return jax.devices("tpu")[0]
    except Exception:
        return jax.devices()[0]


_jitted = {}


def _get_jitted():
    dev = _pick_device()
    fn = _jitted.get(dev)
    if fn is None:
        fmt = jlayout.Format(jlayout.Layout(major_to_minor=(0, 1, 2)),
                             jax.sharding.SingleDeviceSharding(dev))
        fn = jax.jit(_sc_call, out_shardings=fmt)
        _jitted[dev] = fn
    return fn


def kernel(questions, tags, W_question, W_tag, W_pos):
    return _get_jitted()(questions.astype(jnp.int32), tags.astype(jnp.int32),
                         W_question, W_tag, W_pos.reshape(-1))
